# baseline (speedup 1.0000x reference)
import functools

import jax
import jax.numpy as jnp
from jax import lax
from jax.experimental import pallas as pl
from jax.experimental.pallas import tpu as pltpu
from jax.experimental.pallas import tpu_sc as plsc

_N = 10000
_E = 320000
_C = 4
_NFEAT = 128
_HID = 8
_F = _C * _HID
_NCLASS = 32
_ALPHA = 0.2

_NCORE = 2
_NSUB = 16
_NW = _NCORE * _NSUB
_EPW = _E // _NW
_CHUNK = 80
_NCHUNK = _EPW // _CHUNK
_DEPTH = 5
_NTRIP = _NCHUNK // _DEPTH
_NPAD = 10240
_RPT = _NPAD // _NSUB
_DEGW = 8


def _sc_pass_body(*refs):
    (table, edge, z32,
     outp,
     src_v, dst_ring, rows_v, acc_sh, *sems) = refs
    gsem = sems[:_DEPTH]
    isem = sems[_DEPTH:]

    c = lax.axis_index("c")
    s = lax.axis_index("s")
    wid = c * _NSUB + s
    e0 = wid * _EPW

    pltpu.sync_copy(edge.at[0, pl.ds(e0, _EPW)], src_v)

    r0 = s * _RPT
    pltpu.sync_copy(z32.at[pl.ds(r0, _RPT)], acc_sh.at[pl.ds(r0, _RPT)])
    plsc.subcore_barrier()

    for b in range(_DEPTH):
        pltpu.async_copy(
            edge.at[1, pl.ds(e0 + b * _CHUNK, _CHUNK)], dst_ring.at[b],
            isem[b])
        pltpu.async_copy(
            table.at[src_v.at[pl.ds(b * _CHUNK, _CHUNK)]], rows_v.at[b],
            gsem[b])

    def trip(t, carry):
        base = t * _DEPTH
        for b in range(_DEPTH):
            i = base + b
            pltpu.make_async_copy(
                edge.at[1, pl.ds(e0 + i * _CHUNK, _CHUNK)], dst_ring.at[b],
                isem[b]).wait()
            pltpu.make_async_copy(
                table.at[src_v.at[pl.ds(i * _CHUNK, _CHUNK)]], rows_v.at[b],
                gsem[b]).wait()
            pltpu.sync_copy(rows_v.at[b], acc_sh.at[dst_ring.at[b]], add=True)

            @pl.when(t < _NTRIP - 1)
            def _():
                j = i + _DEPTH
                pltpu.async_copy(
                    edge.at[1, pl.ds(e0 + j * _CHUNK, _CHUNK)], dst_ring.at[b],
                    isem[b])
                pltpu.async_copy(
                    table.at[src_v.at[pl.ds(j * _CHUNK, _CHUNK)]],
                    rows_v.at[b], gsem[b])
        return carry

    lax.fori_loop(0, _NTRIP, trip, 0)
    plsc.subcore_barrier()

    pltpu.sync_copy(acc_sh.at[pl.ds(r0, _RPT)], outp.at[c, pl.ds(r0, _RPT)])


def _make_sc_pass():
    mesh = plsc.VectorSubcoreMesh(core_axis_name="c", subcore_axis_name="s")
    scratch = [
        pltpu.VMEM((_EPW,), jnp.int32),
        pltpu.VMEM((_DEPTH, _CHUNK), jnp.int32),
        pltpu.VMEM((_DEPTH, _CHUNK, _F), jnp.float32),
        pltpu.VMEM_SHARED((_NPAD, _F), jnp.float32),
    ]
    scratch.extend([pltpu.SemaphoreType.DMA] * (2 * _DEPTH))

    return pl.kernel(
        _sc_pass_body,
        mesh=mesh,
        out_type=jax.ShapeDtypeStruct((_NCORE, _NPAD, _F), jnp.float32),
        scratch_types=scratch,
        compiler_params=pltpu.CompilerParams(use_tc_tiling_on_sc=False),
    )


def _sc_deg_body(edge, z8, ones_h, degp, dst_ring, ones_v, deg_sh, *isem):
    c = lax.axis_index("c")
    s = lax.axis_index("s")
    wid = c * _NSUB + s
    e0 = wid * _EPW

    r0 = s * _RPT
    pltpu.sync_copy(z8.at[pl.ds(r0, _RPT)], deg_sh.at[pl.ds(r0, _RPT)])
    pltpu.sync_copy(ones_h, ones_v)
    plsc.subcore_barrier()

    for b in range(_DEPTH):
        pltpu.async_copy(
            edge.at[1, pl.ds(e0 + b * _CHUNK, _CHUNK)], dst_ring.at[b],
            isem[b])

    def trip(t, carry):
        base = t * _DEPTH
        for b in range(_DEPTH):
            i = base + b
            pltpu.make_async_copy(
                edge.at[1, pl.ds(e0 + i * _CHUNK, _CHUNK)], dst_ring.at[b],
                isem[b]).wait()
            pltpu.sync_copy(ones_v, deg_sh.at[dst_ring.at[b]], add=True)

            @pl.when(t < _NTRIP - 1)
            def _():
                pltpu.async_copy(
                    edge.at[1, pl.ds(e0 + (i + _DEPTH) * _CHUNK, _CHUNK)],
                    dst_ring.at[b], isem[b])
        return carry

    lax.fori_loop(0, _NTRIP, trip, 0)
    plsc.subcore_barrier()
    pltpu.sync_copy(deg_sh.at[pl.ds(r0, _RPT)], degp.at[c, pl.ds(r0, _RPT)])


def _make_sc_deg():
    mesh = plsc.VectorSubcoreMesh(core_axis_name="c", subcore_axis_name="s")
    scratch = [
        pltpu.VMEM((_DEPTH, _CHUNK), jnp.int32),
        pltpu.VMEM((_CHUNK, _DEGW), jnp.float32),
        pltpu.VMEM_SHARED((_NPAD, _DEGW), jnp.float32),
    ]
    scratch.extend([pltpu.SemaphoreType.DMA] * _DEPTH)

    return pl.kernel(
        _sc_deg_body,
        mesh=mesh,
        out_type=jax.ShapeDtypeStruct((_NCORE, _NPAD, _DEGW), jnp.float32),
        scratch_types=scratch,
        compiler_params=pltpu.CompilerParams(use_tc_tiling_on_sc=False),
    )


_ROWS = 2000


def _tc1_body(x_ref, wl_ref, wr_ref, b_ref, y_ref, z_ref):
    y = jnp.zeros((_ROWS, _F), jnp.float32)
    z = jnp.zeros((_ROWS, _F), jnp.float32)
    for k in range(_C):
        xk = x_ref[k]
        y = y + jnp.dot(xk, wl_ref[k], preferred_element_type=jnp.float32)
        z = z + jnp.dot(xk, wr_ref[k], preferred_element_type=jnp.float32)
    y_ref[...] = y
    z_ref[...] = z + b_ref[...]


def _tc2_body(p_ref, d_ref, z_ref, wl_ref, wr_ref, b2_ref, u_ref, v_ref):
    agg = p_ref[0] + p_ref[1]
    deg = d_ref[0, :, 0:1] + d_ref[1, :, 0:1]
    t = agg / jnp.maximum(deg, 1.0) + z_ref[...]
    h = jnp.where(t > 0, t, _ALPHA * (jnp.exp(t) - 1.0))
    u_ref[...] = jnp.dot(h, wl_ref[...], preferred_element_type=jnp.float32)
    v_ref[...] = (jnp.dot(h, wr_ref[...], preferred_element_type=jnp.float32)
                  + b2_ref[...])


def _tc3_body(p_ref, d_ref, v_ref, o_ref):
    agg = p_ref[0] + p_ref[1]
    deg = d_ref[0, :, 0:1] + d_ref[1, :, 0:1]
    o_ref[...] = agg / jnp.maximum(deg, 1.0) + v_ref[...]


def _row_spec(shape, rows=_ROWS):
    if len(shape) == 3:
        return pl.BlockSpec((shape[0], rows, shape[2]), lambda i: (0, i, 0))
    return pl.BlockSpec((rows, shape[1]), lambda i: (i, 0))


def _full_spec(shape):
    return pl.BlockSpec(shape, lambda i: tuple(0 for _ in shape))


def _tc1(x_list, wl_bd, wr_bd, b1cat):
    grid = (_N // _ROWS,)
    return pl.pallas_call(
        _tc1_body,
        grid=grid,
        in_specs=[
            _row_spec((_C, _N, _NFEAT)),
            _full_spec((_C, _NFEAT, _F)),
            _full_spec((_C, _NFEAT, _F)),
            _full_spec((1, _F)),
        ],
        out_specs=[_row_spec((_N, _F)), _row_spec((_N, _F))],
        out_shape=[jax.ShapeDtypeStruct((_N, _F), jnp.float32)] * 2,
    )(x_list, wl_bd, wr_bd, b1cat)


def _tc2(p, degp, Z, W2l, W2r, b2row):
    return pl.pallas_call(
        _tc2_body,
        grid=(_N // _ROWS,),
        in_specs=[
            _row_spec((_NCORE, _NPAD, _F)),
            _row_spec((_NCORE, _NPAD, _DEGW)),
            _row_spec((_N, _F)),
            _full_spec((_F, _NCLASS)),
            _full_spec((_F, _NCLASS)),
            _full_spec((1, _NCLASS)),
        ],
        out_specs=[_row_spec((_N, _NCLASS))] * 2,
        out_shape=[jax.ShapeDtypeStruct((_N, _NCLASS), jnp.float32)] * 2,
    )(p, degp, Z, W2l, W2r, b2row)


def _tc3(p, degp, V):
    return pl.pallas_call(
        _tc3_body,
        grid=(_N // _ROWS,),
        in_specs=[
            _row_spec((_NCORE, _NPAD, _NCLASS)),
            _row_spec((_NCORE, _NPAD, _DEGW)),
            _row_spec((_N, _NCLASS)),
        ],
        out_specs=_row_spec((_N, _NCLASS)),
        out_shape=jax.ShapeDtypeStruct((_N, _NCLASS), jnp.float32),
    )(p, degp, V)


def kernel(x_list, edge_index, W1l, W1r, b1, W2l, W2r, b2):
    eye = jnp.eye(_C, dtype=jnp.float32)
    wl_bd = jnp.einsum("kfh,kc->kfch", W1l, eye).reshape(_C, _NFEAT, _F)
    wr_bd = jnp.einsum("kfh,kc->kfch", W1r, eye).reshape(_C, _NFEAT, _F)
    b1cat = b1.reshape(1, _F)

    z32 = jnp.zeros((_NPAD, _F), jnp.float32)
    z8 = jnp.zeros((_NPAD, _DEGW), jnp.float32)
    ones8 = jnp.ones((_CHUNK, _DEGW), jnp.float32)

    deg_p = _make_sc_deg()(edge_index, z8, ones8)
    Y, Z = _tc1(x_list, wl_bd, wr_bd, b1cat)
    aggY_p = _make_sc_pass()(Y, edge_index, z32)
    U, V = _tc2(aggY_p, deg_p, Z, W2l, W2r, b2.reshape(1, _NCLASS))
    aggU_p = _make_sc_pass()(U, edge_index, z32)
    return _tc3(aggU_p, deg_p, V)


# speedup vs baseline: 1.0738x; 1.0738x over previous
import functools

import jax
import jax.numpy as jnp
from jax import lax
from jax.experimental import pallas as pl
from jax.experimental.pallas import tpu as pltpu
from jax.experimental.pallas import tpu_sc as plsc

_N = 10000
_E = 320000
_C = 4
_NFEAT = 128
_HID = 8
_F = _C * _HID
_NCLASS = 32
_ALPHA = 0.2

_NCORE = 2
_NSUB = 16
_NW = _NCORE * _NSUB
_EPW = _E // _NW
_CHUNK = 80
_NCHUNK = _EPW // _CHUNK
_DEPTH = 5
_NTRIP = _NCHUNK // _DEPTH
_NPAD = 10240
_RPT = _NPAD // _NSUB
_DEGW = 8


def _sc_pass_body(with_deg, *refs):
    if with_deg:
        (table, edge, z32, z8, ones_h,
         outp, degp,
         src_v, dst_ring, rows_v, ones_v, acc_sh, deg_sh, *sems) = refs
    else:
        (table, edge, z32,
         outp,
         src_v, dst_ring, rows_v, acc_sh, *sems) = refs
    gsem = sems[:_DEPTH]
    isem = sems[_DEPTH:2 * _DEPTH]
    dsem = sems[2 * _DEPTH:]

    c = lax.axis_index("c")
    s = lax.axis_index("s")
    wid = c * _NSUB + s
    e0 = wid * _EPW

    pltpu.sync_copy(edge.at[0, pl.ds(e0, _EPW)], src_v)

    r0 = s * _RPT
    pltpu.sync_copy(z32.at[pl.ds(r0, _RPT)], acc_sh.at[pl.ds(r0, _RPT)])
    if with_deg:
        pltpu.sync_copy(z8.at[pl.ds(r0, _RPT)], deg_sh.at[pl.ds(r0, _RPT)])
        pltpu.sync_copy(ones_h, ones_v)
    plsc.subcore_barrier()

    for b in range(_DEPTH):
        pltpu.async_copy(
            edge.at[1, pl.ds(e0 + b * _CHUNK, _CHUNK)], dst_ring.at[b],
            isem[b])
        pltpu.async_copy(
            table.at[src_v.at[pl.ds(b * _CHUNK, _CHUNK)]], rows_v.at[b],
            gsem[b])

    def trip(t, carry):
        base = t * _DEPTH
        for b in range(_DEPTH):
            i = base + b
            pltpu.make_async_copy(
                edge.at[1, pl.ds(e0 + i * _CHUNK, _CHUNK)], dst_ring.at[b],
                isem[b]).wait()
            if with_deg:
                pltpu.async_copy(
                    ones_v, deg_sh.at[dst_ring.at[b]], dsem[b], add=True)
            pltpu.make_async_copy(
                table.at[src_v.at[pl.ds(i * _CHUNK, _CHUNK)]], rows_v.at[b],
                gsem[b]).wait()
            pltpu.sync_copy(rows_v.at[b], acc_sh.at[dst_ring.at[b]], add=True)
            if with_deg:
                pltpu.make_async_copy(
                    ones_v, deg_sh.at[dst_ring.at[b]], dsem[b]).wait()

            @pl.when(t < _NTRIP - 1)
            def _():
                j = i + _DEPTH
                pltpu.async_copy(
                    edge.at[1, pl.ds(e0 + j * _CHUNK, _CHUNK)], dst_ring.at[b],
                    isem[b])
                pltpu.async_copy(
                    table.at[src_v.at[pl.ds(j * _CHUNK, _CHUNK)]],
                    rows_v.at[b], gsem[b])
        return carry

    lax.fori_loop(0, _NTRIP, trip, 0)
    plsc.subcore_barrier()

    pltpu.sync_copy(acc_sh.at[pl.ds(r0, _RPT)], outp.at[c, pl.ds(r0, _RPT)])
    if with_deg:
        pltpu.sync_copy(deg_sh.at[pl.ds(r0, _RPT)], degp.at[c, pl.ds(r0, _RPT)])


def _make_sc_pass(with_deg):
    mesh = plsc.VectorSubcoreMesh(core_axis_name="c", subcore_axis_name="s")
    out_type = [jax.ShapeDtypeStruct((_NCORE, _NPAD, _F), jnp.float32)]
    scratch = [
        pltpu.VMEM((_EPW,), jnp.int32),
        pltpu.VMEM((_DEPTH, _CHUNK), jnp.int32),
        pltpu.VMEM((_DEPTH, _CHUNK, _F), jnp.float32),
    ]
    if with_deg:
        out_type.append(jax.ShapeDtypeStruct((_NCORE, _NPAD, _DEGW), jnp.float32))
        scratch.append(pltpu.VMEM((_CHUNK, _DEGW), jnp.float32))
    scratch.append(pltpu.VMEM_SHARED((_NPAD, _F), jnp.float32))
    if with_deg:
        scratch.append(pltpu.VMEM_SHARED((_NPAD, _DEGW), jnp.float32))
    nsem = 3 * _DEPTH if with_deg else 2 * _DEPTH
    scratch.extend([pltpu.SemaphoreType.DMA] * nsem)

    return pl.kernel(
        functools.partial(_sc_pass_body, with_deg),
        mesh=mesh,
        out_type=out_type if with_deg else out_type[0],
        scratch_types=scratch,
        compiler_params=pltpu.CompilerParams(use_tc_tiling_on_sc=False),
    )


_ROWS = 2000


def _tc1_body(x_ref, wl_ref, wr_ref, b_ref, y_ref, z_ref):
    y = jnp.zeros((_ROWS, _F), jnp.float32)
    z = jnp.zeros((_ROWS, _F), jnp.float32)
    for k in range(_C):
        xk = x_ref[k]
        y = y + jnp.dot(xk, wl_ref[k], preferred_element_type=jnp.float32)
        z = z + jnp.dot(xk, wr_ref[k], preferred_element_type=jnp.float32)
    y_ref[...] = y
    z_ref[...] = z + b_ref[...]


def _tc2_body(p_ref, d_ref, z_ref, wl_ref, wr_ref, b2_ref, u_ref, v_ref):
    agg = p_ref[0] + p_ref[1]
    deg = d_ref[0, :, 0:1] + d_ref[1, :, 0:1]
    t = agg / jnp.maximum(deg, 1.0) + z_ref[...]
    h = jnp.where(t > 0, t, _ALPHA * (jnp.exp(t) - 1.0))
    u_ref[...] = jnp.dot(h, wl_ref[...], preferred_element_type=jnp.float32)
    v_ref[...] = (jnp.dot(h, wr_ref[...], preferred_element_type=jnp.float32)
                  + b2_ref[...])


def _tc3_body(p_ref, d_ref, v_ref, o_ref):
    agg = p_ref[0] + p_ref[1]
    deg = d_ref[0, :, 0:1] + d_ref[1, :, 0:1]
    o_ref[...] = agg / jnp.maximum(deg, 1.0) + v_ref[...]


def _row_spec(shape, rows=_ROWS):
    if len(shape) == 3:
        return pl.BlockSpec((shape[0], rows, shape[2]), lambda i: (0, i, 0))
    return pl.BlockSpec((rows, shape[1]), lambda i: (i, 0))


def _full_spec(shape):
    return pl.BlockSpec(shape, lambda i: tuple(0 for _ in shape))


def _tc1(x_list, wl_bd, wr_bd, b1cat):
    grid = (_N // _ROWS,)
    return pl.pallas_call(
        _tc1_body,
        grid=grid,
        in_specs=[
            _row_spec((_C, _N, _NFEAT)),
            _full_spec((_C, _NFEAT, _F)),
            _full_spec((_C, _NFEAT, _F)),
            _full_spec((1, _F)),
        ],
        out_specs=[_row_spec((_N, _F)), _row_spec((_N, _F))],
        out_shape=[jax.ShapeDtypeStruct((_N, _F), jnp.float32)] * 2,
    )(x_list, wl_bd, wr_bd, b1cat)


def _tc2(p, degp, Z, W2l, W2r, b2row):
    return pl.pallas_call(
        _tc2_body,
        grid=(_N // _ROWS,),
        in_specs=[
            _row_spec((_NCORE, _NPAD, _F)),
            _row_spec((_NCORE, _NPAD, _DEGW)),
            _row_spec((_N, _F)),
            _full_spec((_F, _NCLASS)),
            _full_spec((_F, _NCLASS)),
            _full_spec((1, _NCLASS)),
        ],
        out_specs=[_row_spec((_N, _NCLASS))] * 2,
        out_shape=[jax.ShapeDtypeStruct((_N, _NCLASS), jnp.float32)] * 2,
    )(p, degp, Z, W2l, W2r, b2row)


def _tc3(p, degp, V):
    return pl.pallas_call(
        _tc3_body,
        grid=(_N // _ROWS,),
        in_specs=[
            _row_spec((_NCORE, _NPAD, _NCLASS)),
            _row_spec((_NCORE, _NPAD, _DEGW)),
            _row_spec((_N, _NCLASS)),
        ],
        out_specs=_row_spec((_N, _NCLASS)),
        out_shape=jax.ShapeDtypeStruct((_N, _NCLASS), jnp.float32),
    )(p, degp, V)


def kernel(x_list, edge_index, W1l, W1r, b1, W2l, W2r, b2):
    eye = jnp.eye(_C, dtype=jnp.float32)
    wl_bd = jnp.einsum("kfh,kc->kfch", W1l, eye).reshape(_C, _NFEAT, _F)
    wr_bd = jnp.einsum("kfh,kc->kfch", W1r, eye).reshape(_C, _NFEAT, _F)
    b1cat = b1.reshape(1, _F)

    z32 = jnp.zeros((_NPAD, _F), jnp.float32)
    z8 = jnp.zeros((_NPAD, _DEGW), jnp.float32)
    ones8 = jnp.ones((_CHUNK, _DEGW), jnp.float32)

    Y, Z = _tc1(x_list, wl_bd, wr_bd, b1cat)
    aggY_p, deg_p = _make_sc_pass(True)(Y, edge_index, z32, z8, ones8)
    U, V = _tc2(aggY_p, deg_p, Z, W2l, W2r, b2.reshape(1, _NCLASS))
    aggU_p = _make_sc_pass(False)(U, edge_index, z32)
    return _tc3(aggU_p, deg_p, V)


# speedup vs baseline: 1.1194x; 1.0424x over previous
import functools

import jax
import jax.numpy as jnp
from jax import lax
from jax.experimental import pallas as pl
from jax.experimental.pallas import tpu as pltpu
from jax.experimental.pallas import tpu_sc as plsc

_N = 10000
_E = 320000
_C = 4
_NFEAT = 128
_HID = 8
_F = _C * _HID
_NCLASS = 32
_ALPHA = 0.2

_NCORE = 2
_NSUB = 16
_NW = _NCORE * _NSUB
_EPW = _E // _NW
_CHUNK = 80
_NCHUNK = _EPW // _CHUNK
_DEPTH = 5
_NTRIP = _NCHUNK // _DEPTH
_NPAD = 10240
_RPT = _NPAD // _NSUB
_DEGW = 16
_HROWS = _NPAD // _NW


def _sc_pass_body(with_deg, *refs):
    if with_deg:
        (table, edge, z32, z16, ones_h,
         outp, degp,
         src_v, dst_ring, rows_v, ones_v, acc_sh, deg_sh, *sems) = refs
    else:
        (table, edge, z32,
         outp,
         src_v, dst_ring, rows_v, acc_sh, *sems) = refs
    gsem = sems[:_DEPTH]
    isem = sems[_DEPTH:2 * _DEPTH]
    dsem = sems[2 * _DEPTH:]

    c = lax.axis_index("c")
    s = lax.axis_index("s")
    wid = c * _NSUB + s
    e0 = wid * _EPW

    pltpu.sync_copy(edge.at[0, pl.ds(e0, _EPW)], src_v)

    r0 = s * _RPT
    pltpu.sync_copy(z32.at[pl.ds(r0, _RPT)], acc_sh.at[pl.ds(r0, _RPT)])
    if with_deg:
        pltpu.sync_copy(z16.at[pl.ds(r0, _RPT)], deg_sh.at[pl.ds(r0, _RPT)])
        pltpu.sync_copy(ones_h, ones_v)
    plsc.subcore_barrier()

    for b in range(_DEPTH):
        pltpu.async_copy(
            edge.at[1, pl.ds(e0 + b * _CHUNK, _CHUNK)], dst_ring.at[b],
            isem[b])
        pltpu.async_copy(
            table.at[src_v.at[pl.ds(b * _CHUNK, _CHUNK)]], rows_v.at[b],
            gsem[b])

    def trip(t, carry):
        base = t * _DEPTH
        for b in range(_DEPTH):
            i = base + b
            pltpu.make_async_copy(
                edge.at[1, pl.ds(e0 + i * _CHUNK, _CHUNK)], dst_ring.at[b],
                isem[b]).wait()
            if with_deg:
                pltpu.async_copy(
                    ones_v, deg_sh.at[dst_ring.at[b]], dsem[b], add=True)
            pltpu.make_async_copy(
                table.at[src_v.at[pl.ds(i * _CHUNK, _CHUNK)]], rows_v.at[b],
                gsem[b]).wait()
            pltpu.sync_copy(rows_v.at[b], acc_sh.at[dst_ring.at[b]], add=True)
            if with_deg:
                pltpu.make_async_copy(
                    ones_v, deg_sh.at[dst_ring.at[b]], dsem[b]).wait()

            @pl.when(t < _NTRIP - 1)
            def _():
                j = i + _DEPTH
                pltpu.async_copy(
                    edge.at[1, pl.ds(e0 + j * _CHUNK, _CHUNK)], dst_ring.at[b],
                    isem[b])
                pltpu.async_copy(
                    table.at[src_v.at[pl.ds(j * _CHUNK, _CHUNK)]],
                    rows_v.at[b], gsem[b])
        return carry

    lax.fori_loop(0, _NTRIP, trip, 0)
    plsc.subcore_barrier()

    pltpu.sync_copy(acc_sh.at[pl.ds(r0, _RPT)], outp.at[c, pl.ds(r0, _RPT)])
    if with_deg:
        pltpu.sync_copy(deg_sh.at[pl.ds(r0, _RPT)], degp.at[c, pl.ds(r0, _RPT)])


def _make_sc_pass(with_deg):
    mesh = plsc.VectorSubcoreMesh(core_axis_name="c", subcore_axis_name="s")
    out_type = [jax.ShapeDtypeStruct((_NCORE, _NPAD, _F), jnp.float32)]
    scratch = [
        pltpu.VMEM((_EPW,), jnp.int32),
        pltpu.VMEM((_DEPTH, _CHUNK), jnp.int32),
        pltpu.VMEM((_DEPTH, _CHUNK, _F), jnp.float32),
    ]
    if with_deg:
        out_type.append(jax.ShapeDtypeStruct((_NCORE, _NPAD, _DEGW), jnp.float32))
        scratch.append(pltpu.VMEM((_CHUNK, _DEGW), jnp.float32))
    scratch.append(pltpu.VMEM_SHARED((_NPAD, _F), jnp.float32))
    if with_deg:
        scratch.append(pltpu.VMEM_SHARED((_NPAD, _DEGW), jnp.float32))
    nsem = 3 * _DEPTH if with_deg else 2 * _DEPTH
    scratch.extend([pltpu.SemaphoreType.DMA] * nsem)

    return pl.kernel(
        functools.partial(_sc_pass_body, with_deg),
        mesh=mesh,
        out_type=out_type if with_deg else out_type[0],
        scratch_types=scratch,
        compiler_params=pltpu.CompilerParams(use_tc_tiling_on_sc=False),
    )


def _sc_h_body(p1, degp, zp, h_out, p0v, p1v, d0v, d1v, zv, hv):
    c = lax.axis_index("c")
    s = lax.axis_index("s")
    w = c * _NSUB + s
    r0 = w * _HROWS
    pltpu.sync_copy(p1.at[0, pl.ds(r0, _HROWS)], p0v)
    pltpu.sync_copy(p1.at[1, pl.ds(r0, _HROWS)], p1v)
    pltpu.sync_copy(degp.at[0, pl.ds(r0, _HROWS)], d0v)
    pltpu.sync_copy(degp.at[1, pl.ds(r0, _HROWS)], d1v)
    pltpu.sync_copy(zp.at[pl.ds(r0, _HROWS)], zv)

    def rows(g, carry):
        for u in range(4):
            r = 4 * g + u
            rdeg = 1.0 / jnp.maximum(d0v[r] + d1v[r], 1.0)
            for half in range(2):
                sl = pl.ds(16 * half, 16)
                t = (p0v[r, sl] + p1v[r, sl]) * rdeg + zv[r, sl]
                hv[r, sl] = jnp.where(t > 0.0, t, _ALPHA * (jnp.exp(t) - 1.0))
        return carry

    lax.fori_loop(0, _HROWS // 4, rows, 0)
    pltpu.sync_copy(hv, h_out.at[pl.ds(r0, _HROWS)])


def _make_sc_h():
    mesh = plsc.VectorSubcoreMesh(core_axis_name="c", subcore_axis_name="s")
    scratch = [
        pltpu.VMEM((_HROWS, _F), jnp.float32),
        pltpu.VMEM((_HROWS, _F), jnp.float32),
        pltpu.VMEM((_HROWS, _DEGW), jnp.float32),
        pltpu.VMEM((_HROWS, _DEGW), jnp.float32),
        pltpu.VMEM((_HROWS, _F), jnp.float32),
        pltpu.VMEM((_HROWS, _F), jnp.float32),
    ]
    return pl.kernel(
        _sc_h_body,
        mesh=mesh,
        out_type=jax.ShapeDtypeStruct((_NPAD, _F), jnp.float32),
        scratch_types=scratch,
        compiler_params=pltpu.CompilerParams(use_tc_tiling_on_sc=False),
    )


_ROWS = 2000


def _tc1_body(x_ref, wl_ref, wr_ref, b_ref, y_ref, z_ref):
    y = jnp.zeros((_ROWS, _F), jnp.float32)
    z = jnp.zeros((_ROWS, _F), jnp.float32)
    for k in range(_C):
        xk = x_ref[k]
        y = y + jnp.dot(xk, wl_ref[k], preferred_element_type=jnp.float32)
        z = z + jnp.dot(xk, wr_ref[k], preferred_element_type=jnp.float32)
    y_ref[...] = y
    z_ref[...] = z + b_ref[...]


def _tc2_body(p_ref, d_ref, z_ref, wl_ref, wr_ref, b2_ref, u_ref, v_ref):
    agg = p_ref[0] + p_ref[1]
    deg = d_ref[0, :, 0:1] + d_ref[1, :, 0:1]
    t = agg / jnp.maximum(deg, 1.0) + z_ref[...]
    h = jnp.where(t > 0, t, _ALPHA * (jnp.exp(t) - 1.0))
    u_ref[...] = jnp.dot(h, wl_ref[...], preferred_element_type=jnp.float32)
    v_ref[...] = (jnp.dot(h, wr_ref[...], preferred_element_type=jnp.float32)
                  + b2_ref[...])


def _tcv_body(h_ref, wr_ref, b2_ref, v_ref):
    v_ref[...] = (jnp.dot(h_ref[...], wr_ref[...],
                          preferred_element_type=jnp.float32) + b2_ref[...])


def _tc3_body(p_ref, d_ref, v_ref, wl_ref, o_ref):
    agg = p_ref[0] + p_ref[1]
    deg = d_ref[0, :, 0:1] + d_ref[1, :, 0:1]
    o_ref[...] = (jnp.dot(agg / jnp.maximum(deg, 1.0), wl_ref[...],
                          preferred_element_type=jnp.float32) + v_ref[...])


def _row_spec(shape, rows=_ROWS):
    if len(shape) == 3:
        return pl.BlockSpec((shape[0], rows, shape[2]), lambda i: (0, i, 0))
    return pl.BlockSpec((rows, shape[1]), lambda i: (i, 0))


def _full_spec(shape):
    return pl.BlockSpec(shape, lambda i: tuple(0 for _ in shape))


def _tc1(x_list, wl_bd, wr_bd, b1cat):
    grid = (_N // _ROWS,)
    return pl.pallas_call(
        _tc1_body,
        grid=grid,
        in_specs=[
            _row_spec((_C, _N, _NFEAT)),
            _full_spec((_C, _NFEAT, _F)),
            _full_spec((_C, _NFEAT, _F)),
            _full_spec((1, _F)),
        ],
        out_specs=[_row_spec((_N, _F)), _row_spec((_N, _F))],
        out_shape=[jax.ShapeDtypeStruct((_N, _F), jnp.float32)] * 2,
    )(x_list, wl_bd, wr_bd, b1cat)


def _tcv(h, W2r, b2row):
    return pl.pallas_call(
        _tcv_body,
        grid=(_N // _ROWS,),
        in_specs=[
            _row_spec((_NPAD, _F)),
            _full_spec((_F, _NCLASS)),
            _full_spec((1, _NCLASS)),
        ],
        out_specs=_row_spec((_N, _NCLASS)),
        out_shape=jax.ShapeDtypeStruct((_N, _NCLASS), jnp.float32),
    )(h, W2r, b2row)


def _tc3(p, degp, V, W2l):
    return pl.pallas_call(
        _tc3_body,
        grid=(_N // _ROWS,),
        in_specs=[
            _row_spec((_NCORE, _NPAD, _NCLASS)),
            _row_spec((_NCORE, _NPAD, _DEGW)),
            _row_spec((_N, _NCLASS)),
            _full_spec((_F, _NCLASS)),
        ],
        out_specs=_row_spec((_N, _NCLASS)),
        out_shape=jax.ShapeDtypeStruct((_N, _NCLASS), jnp.float32),
    )(p, degp, V, W2l)


def kernel(x_list, edge_index, W1l, W1r, b1, W2l, W2r, b2):
    eye = jnp.eye(_C, dtype=jnp.float32)
    wl_bd = jnp.einsum("kfh,kc->kfch", W1l, eye).reshape(_C, _NFEAT, _F)
    wr_bd = jnp.einsum("kfh,kc->kfch", W1r, eye).reshape(_C, _NFEAT, _F)
    b1cat = b1.reshape(1, _F)

    z32 = jnp.zeros((_NPAD, _F), jnp.float32)
    z16 = jnp.zeros((_NPAD, _DEGW), jnp.float32)
    ones16 = jnp.ones((_CHUNK, _DEGW), jnp.float32)

    Y, Z = _tc1(x_list, wl_bd, wr_bd, b1cat)
    Zp = jnp.pad(Z, ((0, _NPAD - _N), (0, 0)))
    aggY_p, deg_p = _make_sc_pass(True)(Y, edge_index, z32, z16, ones16)
    h = _make_sc_h()(aggY_p, deg_p, Zp)
    V = _tcv(h, W2r, b2.reshape(1, _NCLASS))
    aggH_p = _make_sc_pass(False)(h, edge_index, z32)
    return _tc3(aggH_p, deg_p, V, W2l)
